# Initial kernel scaffold; baseline (speedup 1.0000x reference)
import jax
import jax.numpy as jnp
from jax.experimental import pallas as pl


def kernel(x, edge_index, W1, b1, W2, b2):
    raise NotImplementedError("write your pallas kernel here")


# speedup vs baseline: 15.0579x; 15.0579x over previous
import functools

import jax
import jax.numpy as jnp
from jax import lax
from jax.experimental import pallas as pl
from jax.experimental.pallas import tpu as pltpu
from jax.experimental.pallas import tpu_sc as plsc

NC = 2
NS = 16
NW = NC * NS
LANES = 16
CHUNK = 128
CW = 16
ZROWS = 128
D = 64


def _mesh():
    return plsc.VectorSubcoreMesh(
        core_axis_name="c", subcore_axis_name="s", num_cores=NC, num_subcores=NS
    )


def _zero_fill(ref, nrows, width):
    def body(i, _):
        for k in range(width // LANES):
            ref[i, pl.ds(k * LANES, LANES)] = jnp.zeros((LANES,), jnp.float32)
        return 0
    lax.fori_loop(0, nrows, body, 0)


def _make_count(n2, cpt):
    rpt = n2 // NS

    @functools.partial(
        pl.kernel,
        out_type=jax.ShapeDtypeStruct((NC, n2, CW), jnp.float32),
        mesh=_mesh(),
        compiler_params=pltpu.CompilerParams(use_tc_tiling_on_sc=False),
        scratch_types=[
            pltpu.VMEM((cpt, 1, CHUNK), jnp.int32),
            pltpu.VMEM((CHUNK, CW), jnp.float32),
            pltpu.VMEM((ZROWS, CW), jnp.float32),
            pltpu.VMEM_SHARED((n2, CW), jnp.float32),
        ],
    )
    def count_kernel(dst_hbm, out_hbm, dstv, onesv, zerov, acc):
        c = lax.axis_index("c")
        s = lax.axis_index("s")

        def fill_ones(i, _):
            onesv[i, :] = jnp.full((LANES,), 1.0 / CW, jnp.float32)
            return 0
        lax.fori_loop(0, CHUNK, fill_ones, 0)
        _zero_fill(zerov, ZROWS, CW)
        for k in range(rpt // ZROWS):
            pltpu.sync_copy(zerov, acc.at[pl.ds(s * rpt + k * ZROWS, ZROWS)])
        plsc.subcore_barrier()

        tile_row0 = (c * NS + s) * cpt
        pltpu.sync_copy(dst_hbm.at[pl.ds(tile_row0, cpt)], dstv)

        def body(j, _):
            pltpu.sync_copy(onesv, acc.at[dstv.at[j, 0]], add=True)
            return 0
        lax.fori_loop(0, cpt, body, 0)
        plsc.subcore_barrier()

        pltpu.sync_copy(
            acc.at[pl.ds(s * rpt, rpt)], out_hbm.at[c, pl.ds(s * rpt, rpt)]
        )

    return count_kernel


def _make_scatter(n2, cpt):
    rpt = n2 // NS

    @functools.partial(
        pl.kernel,
        out_type=jax.ShapeDtypeStruct((NC, n2, D), jnp.float32),
        mesh=_mesh(),
        compiler_params=pltpu.CompilerParams(use_tc_tiling_on_sc=False),
        scratch_types=[
            pltpu.VMEM((cpt, 1, CHUNK), jnp.int32),
            pltpu.VMEM((cpt, 1, CHUNK), jnp.int32),
            pltpu.VMEM((CHUNK, D), jnp.float32),
            pltpu.VMEM((CHUNK, D), jnp.float32),
            pltpu.VMEM((ZROWS, D), jnp.float32),
            pltpu.VMEM_SHARED((n2, D), jnp.float32),
            pltpu.SemaphoreType.DMA,
            pltpu.SemaphoreType.DMA,
        ],
    )
    def scatter_kernel(src_hbm, dst_hbm, g_hbm, out_hbm,
                       srcv, dstv, r0, r1, zerov, acc, sem0, sem1):
        c = lax.axis_index("c")
        s = lax.axis_index("s")

        _zero_fill(zerov, ZROWS, D)
        for k in range(rpt // ZROWS):
            pltpu.sync_copy(zerov, acc.at[pl.ds(s * rpt + k * ZROWS, ZROWS)])

        pltpu.sync_copy(src_hbm.at[c, pl.ds(s * cpt, cpt)], srcv)
        pltpu.sync_copy(dst_hbm.at[c, pl.ds(s * cpt, cpt)], dstv)
        plsc.subcore_barrier()

        pltpu.async_copy(g_hbm.at[srcv.at[0, 0]], r0, sem0)
        pltpu.async_copy(g_hbm.at[srcv.at[1, 0]], r1, sem1)

        def body(i, _):
            j0 = 2 * i
            pltpu.make_async_copy(g_hbm.at[pl.ds(0, CHUNK)], r0, sem0).wait()
            pltpu.sync_copy(r0, acc.at[dstv.at[j0, 0]], add=True)
            pltpu.async_copy(g_hbm.at[srcv.at[j0 + 2, 0]], r0, sem0)
            j1 = j0 + 1
            pltpu.make_async_copy(g_hbm.at[pl.ds(0, CHUNK)], r1, sem1).wait()
            pltpu.sync_copy(r1, acc.at[dstv.at[j1, 0]], add=True)
            pltpu.async_copy(g_hbm.at[srcv.at[j1 + 2, 0]], r1, sem1)
            return 0
        lax.fori_loop(0, cpt // 2 - 1, body, 0)

        pltpu.make_async_copy(g_hbm.at[pl.ds(0, CHUNK)], r0, sem0).wait()
        pltpu.sync_copy(r0, acc.at[dstv.at[cpt - 2, 0]], add=True)
        pltpu.make_async_copy(g_hbm.at[pl.ds(0, CHUNK)], r1, sem1).wait()
        pltpu.sync_copy(r1, acc.at[dstv.at[cpt - 1, 0]], add=True)
        plsc.subcore_barrier()

        pltpu.sync_copy(
            acc.at[pl.ds(s * rpt, rpt)], out_hbm.at[c, pl.ds(s * rpt, rpt)]
        )

    return scatter_kernel


def _make_tc_first(n2):
    def tc_first(x_ref, w1_ref, cnt_ref, gcat_ref, dinv_ref):
        deg = 1.0 + jnp.sum(
            cnt_ref[0] + cnt_ref[1], axis=-1, keepdims=True
        )
        dinv = lax.rsqrt(deg)
        h = jnp.dot(x_ref[...], w1_ref[...], preferred_element_type=jnp.float32)
        g = h * dinv
        gcat_ref[pl.ds(0, n2)] = g[:, :D]
        gcat_ref[pl.ds(n2, n2)] = g[:, D:]
        dinv_ref[...] = dinv
    return tc_first


def _make_tc_mid(n2):
    def tc_mid(p_ref, gcat_ref, dinv_ref, b1_ref, w2_ref, g2_ref):
        dinv = dinv_ref[...]
        za = (p_ref[0] + gcat_ref[pl.ds(0, n2)]) * dinv + b1_ref[0][None, :]
        zb = (p_ref[1] + gcat_ref[pl.ds(n2, n2)]) * dinv + b1_ref[1][None, :]
        za = jnp.maximum(za, 0.0)
        zb = jnp.maximum(zb, 0.0)
        g2_ref[...] = (
            jnp.dot(za, w2_ref[0], preferred_element_type=jnp.float32)
            + jnp.dot(zb, w2_ref[1], preferred_element_type=jnp.float32)
        ) * dinv
    return tc_mid


def _make_tc_last(n):
    def tc_last(q_ref, g2_ref, dinv_ref, b2_ref, o_ref):
        q = q_ref[0, :n] + q_ref[1, :n]
        z = (q + g2_ref[pl.ds(0, n)]) * dinv_ref[pl.ds(0, n)] + b2_ref[...][None, :]
        m = jnp.max(z, axis=-1, keepdims=True)
        lse = jnp.log(jnp.sum(jnp.exp(z - m), axis=-1, keepdims=True)) + m
        o_ref[...] = z - lse
    return tc_last


@jax.jit
def kernel(x, edge_index, W1, b1, W2, b2):
    n, f_in = x.shape
    hid = W1.shape[1]
    ncls = W2.shape[1]
    e = edge_index.shape[1]

    unit = NW * CHUNK * 8
    e_pad = -(-e // unit) * unit
    nch = e_pad // CHUNK
    unit_n = NS * ZROWS
    n2 = -(-n // unit_n) * unit_n

    src = jnp.concatenate(
        [edge_index[0], jnp.zeros((e_pad - e,), edge_index.dtype)]
    ).reshape(nch, 1, CHUNK)
    dst = jnp.concatenate(
        [edge_index[1], jnp.full((e_pad - e,), n, edge_index.dtype)]
    ).reshape(nch, 1, CHUNK)

    src_l1 = jnp.stack([src, src + n2])
    dst_l1 = jnp.stack([dst, dst])
    src_l2 = src.reshape(NC, nch // NC, 1, CHUNK)
    dst_l2 = dst.reshape(NC, nch // NC, 1, CHUNK)

    x2 = jnp.pad(x, ((0, n2 - n), (0, 0)))

    cnt = _make_count(n2, nch // NW)(dst)

    gcat, dinv = pl.pallas_call(
        _make_tc_first(n2),
        out_shape=(
            jax.ShapeDtypeStruct((2 * n2, D), jnp.float32),
            jax.ShapeDtypeStruct((n2, 1), jnp.float32),
        ),
    )(x2, W1, cnt)

    p = _make_scatter(n2, nch // NS)(src_l1, dst_l1, gcat)

    g2 = pl.pallas_call(
        _make_tc_mid(n2),
        out_shape=jax.ShapeDtypeStruct((n2, ncls), jnp.float32),
    )(p, gcat, dinv, b1.reshape(2, D), W2.reshape(2, D, ncls))

    q = _make_scatter(n2, nch // NW)(src_l2, dst_l2, g2)

    out = pl.pallas_call(
        _make_tc_last(n),
        out_shape=jax.ShapeDtypeStruct((n, ncls), jnp.float32),
    )(q, g2, dinv, b2)

    return out
